# Initial kernel scaffold; baseline (speedup 1.0000x reference)
import jax
import jax.numpy as jnp
from jax.experimental import pallas as pl


def kernel(hidden_states, router_w, e_score_correction_bias, gate_w, up_w, down_w):
    raise NotImplementedError("write your pallas kernel here")


# speedup vs baseline: 1.5370x; 1.5370x over previous
import functools

import jax
import jax.numpy as jnp
from jax.experimental import pallas as pl
from jax.experimental.pallas import tpu as pltpu

E = 8
TOP_K = 2
D = 768
F = 1536
TM = 256
NT = 24


def _gmm_body(map_ref, act_ref, xs_ref, gw_ref, uw_ref, dw_ref, rw_ref, ys_ref):
    i = pl.program_id(0)

    @pl.when(act_ref[i] != 0)
    def _compute():
        x = xs_ref[...]
        g = jnp.dot(x, gw_ref[0], preferred_element_type=jnp.float32)
        u = jnp.dot(x, uw_ref[0], preferred_element_type=jnp.float32)
        h = g * jax.nn.sigmoid(g) * u
        y = jnp.dot(h, dw_ref[0], preferred_element_type=jnp.float32)
        w = rw_ref[0, 0, :]
        ys_ref[...] = y * w[:, None]

    @pl.when(act_ref[i] == 0)
    def _zero():
        ys_ref[...] = jnp.zeros_like(ys_ref)


def _grouped_mlp(tile_map, tile_act, xs, gate_w, up_w, down_w, row_w):
    grid_spec = pltpu.PrefetchScalarGridSpec(
        num_scalar_prefetch=2,
        grid=(NT,),
        in_specs=[
            pl.BlockSpec((TM, D), lambda i, m, a: (i, 0)),
            pl.BlockSpec((1, D, F), lambda i, m, a: (m[i], 0, 0)),
            pl.BlockSpec((1, D, F), lambda i, m, a: (m[i], 0, 0)),
            pl.BlockSpec((1, F, D), lambda i, m, a: (m[i], 0, 0)),
            pl.BlockSpec((1, 1, TM), lambda i, m, a: (i, 0, 0)),
        ],
        out_specs=pl.BlockSpec((TM, D), lambda i, m, a: (i, 0)),
    )
    return pl.pallas_call(
        _gmm_body,
        grid_spec=grid_spec,
        out_shape=jax.ShapeDtypeStruct((NT * TM, D), jnp.float32),
        compiler_params=pltpu.CompilerParams(
            dimension_semantics=("arbitrary",),
        ),
    )(tile_map, tile_act, xs, gate_w, up_w, down_w, row_w)


def kernel(hidden_states, router_w, e_score_correction_bias, gate_w, up_w, down_w):
    b, s, d = hidden_states.shape
    x = hidden_states.reshape(-1, d)
    T = x.shape[0]
    P = NT * TM

    logits = jnp.dot(x.astype(jnp.float32), router_w.T.astype(jnp.float32))
    scores = jax.nn.sigmoid(logits)
    corrected = scores + e_score_correction_bias[None, :]
    _, topk_idx = jax.lax.top_k(corrected, TOP_K)
    affin = jnp.take_along_axis(scores, topk_idx, axis=1)
    affin = affin / (jnp.sum(affin, axis=-1, keepdims=True) + 1e-9)

    expert_a = topk_idx.T.reshape(-1)
    weight_a = affin.T.reshape(-1).astype(jnp.float32)
    token_a = jnp.concatenate([jnp.arange(T, dtype=jnp.int32)] * TOP_K)
    counts = jnp.bincount(expert_a, length=E)
    tiles_e = (counts + TM - 1) // TM
    cum_tiles = jnp.cumsum(tiles_e)
    tile_start = cum_tiles - tiles_e
    total_tiles = cum_tiles[-1]

    order = jnp.argsort(expert_a, stable=True)
    sorted_e = expert_a[order]
    grp_start = jnp.cumsum(counts) - counts
    pos_sorted = (tile_start[sorted_e] * TM
                  + jnp.arange(T * TOP_K) - grp_start[sorted_e])
    pos = jnp.zeros((T * TOP_K,), jnp.int32).at[order].set(
        pos_sorted.astype(jnp.int32))

    row_token = jnp.zeros((P,), jnp.int32).at[pos].set(token_a)
    row_w = jnp.zeros((P,), jnp.float32).at[pos].set(weight_a)

    t_ids = jnp.arange(NT, dtype=jnp.int32)
    tile_map = jnp.minimum(
        jnp.searchsorted(cum_tiles, t_ids, side="right"), E - 1
    ).astype(jnp.int32)
    tile_act = (t_ids < total_tiles).astype(jnp.int32)

    xs = x[row_token]
    ys = _grouped_mlp(tile_map, tile_act, xs, gate_w, up_w, down_w,
                      row_w.reshape(NT, 1, TM))

    inv1, inv2 = pos[:T], pos[T:]
    out = ys[inv1] + ys[inv2]
    return out.reshape(b, s, d)


# speedup vs baseline: 2.3421x; 1.5238x over previous
import functools

import jax
import jax.numpy as jnp
from jax import lax
from jax.experimental import pallas as pl
from jax.experimental.pallas import tpu as pltpu
from jax.experimental.pallas import tpu_sc as plsc

E = 8
TOP_K = 2
D = 768
F = 1536
T = 2048
A = T * TOP_K
TM = 256
NT = 24
P = NT * TM

NW = 32
CHUNK = A // NW
NG = CHUNK // 16

_MESH = plsc.VectorSubcoreMesh(core_axis_name="c", subcore_axis_name="s")


def _dispatch_body(pos_hbm, x_hbm, xs_hbm, pos_v, tok_v, rows_v, sem):
    wid = lax.axis_index("s") * 2 + lax.axis_index("c")

    pltpu.sync_copy(pos_hbm.at[pl.ds(wid * CHUNK, CHUNK)], pos_v)
    tok_base = (wid % (NW // TOP_K)) * CHUNK
    for g in range(NG):
        tok_v[pl.ds(g * 16, 16)] = (
            lax.iota(jnp.int32, 16) + (tok_base + g * 16))

    pltpu.async_copy(x_hbm.at[tok_v], rows_v, sem).wait()
    pltpu.async_copy(rows_v, xs_hbm.at[pos_v], sem).wait()


@functools.partial(
    pl.kernel,
    out_type=jax.ShapeDtypeStruct((P, D), jnp.float32),
    mesh=_MESH,
    scratch_types=[
        pltpu.VMEM((CHUNK,), jnp.int32),
        pltpu.VMEM((CHUNK,), jnp.int32),
        pltpu.VMEM((CHUNK, D), jnp.float32),
        pltpu.SemaphoreType.DMA,
    ],
)
def _sc_dispatch(pos_hbm, x_hbm, *rest):
    _dispatch_body(pos_hbm, x_hbm, *rest)


TPW = T // NW


def _combine_body(ys_hbm, wa_hbm, inv_hbm, out_hbm, idx1_v, idx2_v, w1_v,
                  w2_v, y1_v, y2_v, sem):
    wid = lax.axis_index("s") * 2 + lax.axis_index("c")
    t0 = wid * TPW

    pltpu.sync_copy(inv_hbm.at[pl.ds(t0, TPW)], idx1_v)
    pltpu.sync_copy(inv_hbm.at[pl.ds(T + t0, TPW)], idx2_v)
    pltpu.sync_copy(wa_hbm.at[pl.ds(t0, TPW)], w1_v)
    pltpu.sync_copy(wa_hbm.at[pl.ds(T + t0, TPW)], w2_v)

    c1 = pltpu.async_copy(ys_hbm.at[idx1_v], y1_v, sem)
    c2 = pltpu.async_copy(ys_hbm.at[idx2_v], y2_v, sem)
    c1.wait()
    c2.wait()

    for g in range(TPW // 16):
        w1g = w1_v[pl.ds(g * 16, 16)]
        w2g = w2_v[pl.ds(g * 16, 16)]
        for i in range(16):
            t = g * 16 + i
            w1s = w1g[i]
            w2s = w2g[i]

            def _vec(v, _, t=t, w1s=w1s, w2s=w2s):
                for j in range(4):
                    sl = pl.ds(v * 64 + j * 16, 16)
                    y1_v[t, sl] = w1s * y1_v[t, sl] + w2s * y2_v[t, sl]
                return 0

            lax.fori_loop(0, D // 64, _vec, 0)
    pltpu.sync_copy(y1_v, out_hbm.at[pl.ds(t0, TPW)])


@functools.partial(
    pl.kernel,
    out_type=jax.ShapeDtypeStruct((T, D), jnp.float32),
    mesh=_MESH,
    scratch_types=[
        pltpu.VMEM((TPW,), jnp.int32),
        pltpu.VMEM((TPW,), jnp.int32),
        pltpu.VMEM((TPW,), jnp.float32),
        pltpu.VMEM((TPW,), jnp.float32),
        pltpu.VMEM((TPW, D), jnp.float32),
        pltpu.VMEM((TPW, D), jnp.float32),
        pltpu.SemaphoreType.DMA,
    ],
)
def _sc_combine(ys_hbm, wa_hbm, inv_hbm, *rest):
    _combine_body(ys_hbm, wa_hbm, inv_hbm, *rest)


def _gmm_body(map_ref, act_ref, xs_ref, gw_ref, uw_ref, dw_ref, ys_ref):
    i = pl.program_id(0)

    @pl.when(act_ref[i] != 0)
    def _compute():
        x = xs_ref[...]
        g = jnp.dot(x, gw_ref[0], preferred_element_type=jnp.float32)
        u = jnp.dot(x, uw_ref[0], preferred_element_type=jnp.float32)
        h = g * jax.nn.sigmoid(g) * u
        ys_ref[...] = jnp.dot(h, dw_ref[0], preferred_element_type=jnp.float32)


def _grouped_mlp(tile_map, tile_act, xs, gate_w, up_w, down_w):
    grid_spec = pltpu.PrefetchScalarGridSpec(
        num_scalar_prefetch=2,
        grid=(NT,),
        in_specs=[
            pl.BlockSpec((TM, D), lambda i, m, a: (i, 0)),
            pl.BlockSpec((1, D, F), lambda i, m, a: (m[i], 0, 0)),
            pl.BlockSpec((1, D, F), lambda i, m, a: (m[i], 0, 0)),
            pl.BlockSpec((1, F, D), lambda i, m, a: (m[i], 0, 0)),
        ],
        out_specs=pl.BlockSpec((TM, D), lambda i, m, a: (i, 0)),
    )
    return pl.pallas_call(
        _gmm_body,
        grid_spec=grid_spec,
        out_shape=jax.ShapeDtypeStruct((P, D), jnp.float32),
        compiler_params=pltpu.CompilerParams(
            dimension_semantics=("arbitrary",),
        ),
    )(tile_map, tile_act, xs, gate_w, up_w, down_w)


def kernel(hidden_states, router_w, e_score_correction_bias, gate_w, up_w, down_w):
    b, s, d = hidden_states.shape
    x = hidden_states.reshape(-1, d)

    logits = jnp.dot(x.astype(jnp.float32), router_w.T.astype(jnp.float32))
    scores = jax.nn.sigmoid(logits)
    corrected = scores + e_score_correction_bias[None, :]
    _, topk_idx = jax.lax.top_k(corrected, TOP_K)
    affin = jnp.take_along_axis(scores, topk_idx, axis=1)
    affin = affin / (jnp.sum(affin, axis=-1, keepdims=True) + 1e-9)

    ia = topk_idx.T.reshape(-1).astype(jnp.int32)
    wa = affin.T.reshape(-1).astype(jnp.float32)

    onehot = (ia[:, None] == jnp.arange(E, dtype=jnp.int32)).astype(jnp.int32)
    csum = jnp.cumsum(onehot, axis=0)
    counts = csum[-1]
    tiles_e = (counts + TM - 1) // TM
    cum_tiles = jnp.cumsum(tiles_e)
    off = (cum_tiles - tiles_e) * TM
    rank = jnp.take_along_axis(csum, ia[:, None], axis=1)[:, 0] - 1
    pos = (off[ia] + rank).astype(jnp.int32)

    t_ids = jnp.arange(32, dtype=jnp.int32)
    tmap = jnp.minimum(
        jnp.searchsorted(cum_tiles, t_ids, side="right"), E - 1
    ).astype(jnp.int32)
    tact = (t_ids < cum_tiles[-1]).astype(jnp.int32)

    xs = _sc_dispatch(pos, x)
    ys = _grouped_mlp(tmap, tact, xs, gate_w, up_w, down_w)
    out = _sc_combine(ys, wa, pos)
    return out.reshape(b, s, d)
